# baseline (speedup 1.0000x reference)
import functools
import math

import jax
import jax.numpy as jnp
from jax import lax
from jax.experimental import pallas as pl
from jax.experimental.pallas import tpu as pltpu
from jax.experimental.pallas import tpu_sc as plsc

LANES = 16
GROUP = 128
NW = 32


def _emb_call(n_per_w, dim, scale):
    n_groups = n_per_w // GROUP
    wdim = 2 * dim
    mesh = plsc.VectorSubcoreMesh(core_axis_name="c", subcore_axis_name="s")

    @functools.partial(
        pl.kernel,
        mesh=mesh,
        out_type=jax.ShapeDtypeStruct((NW * n_per_w // 2, wdim), jnp.float32),
        scratch_types=[
            pltpu.VMEM((n_per_w,), jnp.int32),
            pltpu.VMEM((n_per_w,), jnp.int32),
            pltpu.VMEM((GROUP, wdim), jnp.float32),
            pltpu.VMEM((GROUP, wdim), jnp.float32),
            pltpu.VMEM((GROUP // 2, wdim), jnp.float32),
            pltpu.VMEM((GROUP // 2, wdim), jnp.float32),
            pltpu.SemaphoreType.DMA,
            pltpu.SemaphoreType.DMA,
            pltpu.SemaphoreType.DMA,
            pltpu.SemaphoreType.DMA,
        ],
        compiler_params=pltpu.CompilerParams(needs_layout_passes=False),
    )
    def emb_kernel(idx_hbm, tab_hbm, out_hbm, idx_v, phys_v,
                   g0, g1, st0, st1, sg0, sg1, ss0, ss1):
        nc = 2
        wid = lax.axis_index("s") * nc + lax.axis_index("c")
        pltpu.sync_copy(idx_hbm.at[wid], idx_v)

        def shift_body(i, c):
            for u in range(8):
                sl = pl.ds(i * 8 * LANES + u * LANES, LANES)
                phys_v[sl] = lax.shift_right_logical(idx_v[sl], 1)
            return c

        lax.fori_loop(0, n_per_w // (8 * LANES), shift_body, 0)

        g_bufs = (g0, g1)
        st_bufs = (st0, st1)
        g_sems = (sg0, sg1)
        s_sems = (ss0, ss1)
        out_base = wid * (n_per_w // 2)
        iota = lax.iota(jnp.int32, LANES)

        def g_src(g):
            return tab_hbm.at[phys_v.at[pl.ds(g * GROUP, GROUP)]]

        def out_dst(g):
            return out_hbm.at[pl.ds(out_base + g * (GROUP // 2), GROUP // 2)]

        pltpu.async_copy(g_src(0), g0, sg0)
        pltpu.async_copy(g_src(1), g1, sg1)

        def scale_group(gb, stb, g):
            base = g * GROUP

            def q_body(q, c):
                for s in range(2):
                    r = 2 * q + s
                    tok = plsc.load_gather(
                        idx_v, [jnp.full((LANES,), base + r, jnp.int32)])
                    col0 = (tok & 1) * dim
                    rvec = jnp.full((LANES,), r, jnp.int32)
                    for j in range(dim // LANES):
                        col = col0 + (j * LANES + iota)
                        v = plsc.load_gather(gb, [rvec, col])
                        stb[q, pl.ds(s * dim + j * LANES, LANES)] = v * scale
                return c

            lax.fori_loop(0, GROUP // 2, q_body, 0)

        def outer(g2, carry):
            for p in range(2):
                g = g2 * 2 + p
                gb, stb = g_bufs[p], st_bufs[p]

                pltpu.make_async_copy(g_src(g), gb, g_sems[p]).wait()

                @pl.when(g2 >= 1)
                def _():
                    pltpu.make_async_copy(
                        stb, out_dst(g - 2), s_sems[p]).wait()

                scale_group(gb, stb, g)

                @pl.when(g2 < (n_groups // 2) - 1)
                def _():
                    pltpu.async_copy(g_src(g + 2), gb, g_sems[p])

                pltpu.async_copy(stb, out_dst(g), s_sems[p])
            return carry

        lax.fori_loop(0, n_groups // 2, outer, 0)

        pltpu.make_async_copy(st0, out_dst(n_groups - 2), ss0).wait()
        pltpu.make_async_copy(st1, out_dst(n_groups - 1), ss1).wait()

    return emb_kernel


def kernel(token_ids_batch, embeddings_table):
    b, l = token_ids_batch.shape
    v, d = embeddings_table.shape
    n_total = b * l
    assert n_total % (NW * GROUP) == 0 and v % 2 == 0
    n_per_w = n_total // NW
    assert (n_per_w // GROUP) % 2 == 0
    scale = math.sqrt(d)

    idx = token_ids_batch.astype(jnp.int32).reshape(NW, n_per_w)
    tab2 = embeddings_table.reshape(v // 2, 2 * d)
    out2 = _emb_call(n_per_w, d, scale)(idx, tab2)
    return out2.reshape(b, l, d)
